# baseline (speedup 1.0000x reference)
import functools

import jax
import jax.numpy as jnp
from jax import lax
from jax.experimental import pallas as pl
from jax.experimental.pallas import tpu as pltpu
from jax.experimental.pallas import tpu_sc as plsc

VOCAB = 1000000
EMBED = 32
BATCH = 16384
NC, NS, L = 2, 16, 16
NW = NC * NS
BPW = BATCH // NW
CH = 128
NCH = BPW // CH

NGRP = 4
NT = 7813
GRP_WORDS = NT * 1024
TOTAL_TILES = NGRP * NT
WIN = 16
NWIN = (NT + WIN - 1) // WIN

_mesh = plsc.VectorSubcoreMesh(core_axis_name="c", subcore_axis_name="s")


RWIN = 64
RNB = (NT + RWIN - 1) // RWIN


def _tc_retile_body(o_in, o_out):
    for g in range(NGRP):
        for i in range(RWIN):
            o_out[g, i] = o_in[pl.ds(g * 8, 8), pl.ds(i * 128, 128)]


_tc_retile = pl.pallas_call(
    _tc_retile_body,
    grid=(RNB,),
    in_specs=[pl.BlockSpec((32, RWIN * 128), lambda w: (0, w))],
    out_specs=pl.BlockSpec((NGRP, RWIN, 8, 128), lambda w: (0, w, 0, 0)),
    out_shape=jax.ShapeDtypeStruct((NGRP, NT, 8, 128), jnp.float32),
    compiler_params=pltpu.CompilerParams(
        dimension_semantics=("arbitrary",)),
)

SWIN = 16
SNW = 61


@functools.partial(
    pl.kernel,
    out_type=jax.ShapeDtypeStruct((NGRP, NT, 8, 128), jnp.float32),
    mesh=_mesh,
    compiler_params=pltpu.CompilerParams(needs_layout_passes=False),
    scratch_types=[
        pltpu.VMEM((SWIN, 8, 128), jnp.float32),
        pltpu.VMEM((SWIN, 8, 128), jnp.float32),
        pltpu.VMEM((8, 128), jnp.float32),
        pltpu.SemaphoreType.DMA((2,)),
    ],
)
def _sc_retile(ct_hbm, tc_hbm, cf_hbm, vb0, vb1, tbuf, in_sems):
    wid = lax.axis_index("s") * NC + lax.axis_index("c")
    g = wid % NGRP
    s = wid // NGRP
    t_base = s * (SNW * SWIN)
    vbufs = (vb0, vb1)

    def src_tile(src, t):
        return src.at[pl.ds(pl.multiple_of(g * 8, 8), 8),
                      pl.ds(pl.multiple_of(t * 128, 128), 128)]

    def fire_in(w, slot):
        t0 = t_base + w * SWIN
        for i in range(SWIN):
            pltpu.async_copy(src_tile(ct_hbm, t0 + i),
                             vbufs[slot].at[i], in_sems.at[slot])

    def wait_in(w, slot):
        t0 = t_base + w * SWIN
        for i in range(SWIN):
            pltpu.make_async_copy(src_tile(ct_hbm, t0 + i),
                                  vbufs[slot].at[i],
                                  in_sems.at[slot]).wait()

    def flush_out(w, slot):
        t0 = t_base + w * SWIN
        pltpu.sync_copy(
            vbufs[slot],
            cf_hbm.at[g, pl.ds(pl.multiple_of(t0, SWIN), SWIN)])

    fire_in(0, 0)
    fire_in(1, 1)

    def pair(b, carry):
        for i in range(2):
            w = b * 2 + i
            wait_in(w, i)
            flush_out(w, i)

            @pl.when(w + 2 < SNW)
            def _next():
                fire_in(w + 2, i)
        return carry

    lax.fori_loop(0, SNW // 2, pair, 0)
    wait_in(SNW - 1, (SNW - 1) % 2)
    flush_out(SNW - 1, (SNW - 1) % 2)

    @pl.when(wid < NGRP)
    def _tails():
        for t in range(8 * SNW * SWIN, NT - 1):
            pltpu.sync_copy(src_tile(ct_hbm, t), tbuf)
            pltpu.sync_copy(tbuf, cf_hbm.at[g, t])
        pltpu.sync_copy(
            tc_hbm.at[pl.ds(pl.multiple_of(g * 8, 8), 8), :], tbuf)
        pltpu.sync_copy(tbuf, cf_hbm.at[g, NT - 1])


@functools.partial(
    pl.kernel,
    out_type=jax.ShapeDtypeStruct((BATCH,), jnp.float32),
    mesh=_mesh,
    compiler_params=pltpu.CompilerParams(
        use_tc_tiling_on_sc=False, needs_layout_passes=False),
    scratch_types=[
        pltpu.VMEM((NCH, CH), jnp.int32),
        pltpu.VMEM((NCH, CH), jnp.int32),
        pltpu.VMEM((EMBED, BPW), jnp.float32),
        pltpu.VMEM((EMBED, BPW), jnp.float32),
        pltpu.VMEM((BPW,), jnp.float32),
        pltpu.SemaphoreType.DMA,
    ],
)
def _sc_dots(x_hbm, y_hbm, cf_hbm, of_hbm, dots_hbm, xp, yp, cbuf, obuf, dv,
             sem):
    wid = lax.axis_index("s") * NC + lax.axis_index("c")
    base = wid * BPW
    pltpu.sync_copy(x_hbm.at[pl.ds(wid * NCH, NCH)], xp)
    pltpu.sync_copy(y_hbm.at[pl.ds(wid * NCH, NCH)], yp)
    for j in range(NCH):
        for k in range(CH // L):
            sl = pl.ds(k * L, L)
            vx = xp[j, sl]
            vy = yp[j, sl]
            xp[j, sl] = lax.shift_left(lax.shift_right_logical(vx, 7), 10) \
                + jnp.bitwise_and(vx, 127)
            yp[j, sl] = lax.shift_left(lax.shift_right_logical(vy, 7), 10) \
                + jnp.bitwise_and(vy, 127)

    for j in range(NCH):
        copies = []
        for c in range(EMBED):
            cbase = (c // 8) * GRP_WORDS + (c % 8) * 128
            clen = (NT - 1) * 1024 + 128
            copies.append(
                pltpu.async_copy(
                    cf_hbm.at[pl.ds(cbase, clen)].at[xp.at[j]],
                    cbuf.at[c, pl.ds(j * CH, CH)], sem))
            copies.append(
                pltpu.async_copy(
                    of_hbm.at[pl.ds(cbase, clen)].at[yp.at[j]],
                    obuf.at[c, pl.ds(j * CH, CH)], sem))
        for cp in copies:
            cp.wait()

    def body(g, carry):
        sl = pl.ds(g * L, L)
        acc = cbuf[0, sl] * obuf[0, sl]
        for c in range(1, EMBED):
            acc = acc + cbuf[c, sl] * obuf[c, sl]
        dv[sl] = acc
        return carry

    lax.fori_loop(0, BPW // L, body, 0)
    pltpu.sync_copy(dv, dots_hbm.at[pl.ds(base, BPW)])


def _tc_loss_body(d_ref, o_ref):
    d = d_ref[...]
    neg_abs = -jnp.abs(d)
    ls = jnp.minimum(d, 0.0) - jnp.log(1.0 + jnp.exp(neg_abs))
    o_ref[0, 0] = -jnp.sum(ls) / BATCH


_tc_loss = pl.pallas_call(
    _tc_loss_body,
    out_shape=jax.ShapeDtypeStruct((1, 1), jnp.float32),
    out_specs=pl.BlockSpec(memory_space=pltpu.SMEM),
)


def kernel(x, y, center_weight, out_weight):
    ct = center_weight.T
    ot = out_weight.T
    tpad = ((0, 0), (0, 128 - (VOCAB - (NT - 1) * 128)))
    tcp = jnp.pad(center_weight[(NT - 1) * 128:].T, tpad)
    cf4 = _sc_retile(ct, tcp)
    of4 = _tc_retile(ot)
    cf = cf4.reshape(NGRP * NT * 8 * 128)
    of = of4.reshape(NGRP * NT * 8 * 128)
    x2 = x.reshape(NW * NCH, CH)
    y2 = y.reshape(NW * NCH, CH)
    dots = _sc_dots(x2, y2, cf, of)
    loss = _tc_loss(dots.reshape(BATCH // 128, 128))
    return loss[0, 0]


# speedup vs baseline: 1.0502x; 1.0502x over previous
import functools

import jax
import jax.numpy as jnp
from jax import lax
from jax.experimental import pallas as pl
from jax.experimental.pallas import tpu as pltpu
from jax.experimental.pallas import tpu_sc as plsc

VOCAB = 1000000
EMBED = 32
BATCH = 16384
NC, NS, L = 2, 16, 16
NW = NC * NS
BPW = BATCH // NW
CH = 128
NCH = BPW // CH

NGRP = 4
NT = 7813
GRP_WORDS = NT * 1024
TOTAL_TILES = NGRP * NT
WIN = 16
NWIN = (NT + WIN - 1) // WIN

_mesh = plsc.VectorSubcoreMesh(core_axis_name="c", subcore_axis_name="s")


RWIN = 128
RNB = (NT + RWIN - 1) // RWIN


def _tc_retile_body(o_in, o_out):
    for g in range(NGRP):
        for i in range(RWIN):
            o_out[g, i] = o_in[pl.ds(g * 8, 8), pl.ds(i * 128, 128)]


_tc_retile = pl.pallas_call(
    _tc_retile_body,
    grid=(RNB,),
    in_specs=[pl.BlockSpec((32, RWIN * 128), lambda w: (0, w))],
    out_specs=pl.BlockSpec((NGRP, RWIN, 8, 128), lambda w: (0, w, 0, 0)),
    out_shape=jax.ShapeDtypeStruct((NGRP, NT, 8, 128), jnp.float32),
    compiler_params=pltpu.CompilerParams(
        dimension_semantics=("parallel",)),
)

SWIN = 16
SNW = 61


@functools.partial(
    pl.kernel,
    out_type=jax.ShapeDtypeStruct((NGRP, NT, 8, 128), jnp.float32),
    mesh=_mesh,
    compiler_params=pltpu.CompilerParams(needs_layout_passes=False),
    scratch_types=[
        pltpu.VMEM((SWIN, 8, 128), jnp.float32),
        pltpu.VMEM((SWIN, 8, 128), jnp.float32),
        pltpu.VMEM((8, 128), jnp.float32),
        pltpu.SemaphoreType.DMA((2,)),
    ],
)
def _sc_retile(ct_hbm, tc_hbm, cf_hbm, vb0, vb1, tbuf, in_sems):
    wid = lax.axis_index("s") * NC + lax.axis_index("c")
    g = wid % NGRP
    s = wid // NGRP
    t_base = s * (SNW * SWIN)
    vbufs = (vb0, vb1)

    def src_tile(src, t):
        return src.at[pl.ds(pl.multiple_of(g * 8, 8), 8),
                      pl.ds(pl.multiple_of(t * 128, 128), 128)]

    def fire_in(w, slot):
        t0 = t_base + w * SWIN
        for i in range(SWIN):
            pltpu.async_copy(src_tile(ct_hbm, t0 + i),
                             vbufs[slot].at[i], in_sems.at[slot])

    def wait_in(w, slot):
        t0 = t_base + w * SWIN
        for i in range(SWIN):
            pltpu.make_async_copy(src_tile(ct_hbm, t0 + i),
                                  vbufs[slot].at[i],
                                  in_sems.at[slot]).wait()

    def flush_out(w, slot):
        t0 = t_base + w * SWIN
        pltpu.sync_copy(
            vbufs[slot],
            cf_hbm.at[g, pl.ds(pl.multiple_of(t0, SWIN), SWIN)])

    fire_in(0, 0)
    fire_in(1, 1)

    def pair(b, carry):
        for i in range(2):
            w = b * 2 + i
            wait_in(w, i)
            flush_out(w, i)

            @pl.when(w + 2 < SNW)
            def _next():
                fire_in(w + 2, i)
        return carry

    lax.fori_loop(0, SNW // 2, pair, 0)
    wait_in(SNW - 1, (SNW - 1) % 2)
    flush_out(SNW - 1, (SNW - 1) % 2)

    @pl.when(wid < NGRP)
    def _tails():
        for t in range(8 * SNW * SWIN, NT - 1):
            pltpu.sync_copy(src_tile(ct_hbm, t), tbuf)
            pltpu.sync_copy(tbuf, cf_hbm.at[g, t])
        pltpu.sync_copy(
            tc_hbm.at[pl.ds(pl.multiple_of(g * 8, 8), 8), :], tbuf)
        pltpu.sync_copy(tbuf, cf_hbm.at[g, NT - 1])


@functools.partial(
    pl.kernel,
    out_type=jax.ShapeDtypeStruct((BATCH,), jnp.float32),
    mesh=_mesh,
    compiler_params=pltpu.CompilerParams(
        use_tc_tiling_on_sc=False, needs_layout_passes=False),
    scratch_types=[
        pltpu.VMEM((NCH, CH), jnp.int32),
        pltpu.VMEM((NCH, CH), jnp.int32),
        pltpu.VMEM((EMBED, BPW), jnp.float32),
        pltpu.VMEM((EMBED, BPW), jnp.float32),
        pltpu.VMEM((BPW,), jnp.float32),
        pltpu.SemaphoreType.DMA,
    ],
)
def _sc_dots(x_hbm, y_hbm, cf_hbm, of_hbm, dots_hbm, xp, yp, cbuf, obuf, dv,
             sem):
    wid = lax.axis_index("s") * NC + lax.axis_index("c")
    base = wid * BPW
    pltpu.sync_copy(x_hbm.at[pl.ds(wid * NCH, NCH)], xp)
    pltpu.sync_copy(y_hbm.at[pl.ds(wid * NCH, NCH)], yp)
    for j in range(NCH):
        for k in range(CH // L):
            sl = pl.ds(k * L, L)
            vx = xp[j, sl]
            vy = yp[j, sl]
            xp[j, sl] = lax.shift_left(lax.shift_right_logical(vx, 7), 10) \
                + jnp.bitwise_and(vx, 127)
            yp[j, sl] = lax.shift_left(lax.shift_right_logical(vy, 7), 10) \
                + jnp.bitwise_and(vy, 127)

    def fire(j):
        copies = []
        for c in range(EMBED):
            cbase = (c // 8) * GRP_WORDS + (c % 8) * 128
            clen = (NT - 1) * 1024 + 128
            copies.append(
                pltpu.async_copy(
                    cf_hbm.at[pl.ds(cbase, clen)].at[xp.at[j]],
                    cbuf.at[c, pl.ds(j * CH, CH)], sem))
            copies.append(
                pltpu.async_copy(
                    of_hbm.at[pl.ds(cbase, clen)].at[yp.at[j]],
                    obuf.at[c, pl.ds(j * CH, CH)], sem))
        return copies

    pending = fire(0)
    for j in range(NCH):
        nxt = fire(j + 1) if j + 1 < NCH else []
        for cp in pending:
            cp.wait()
        pending = nxt

    def body(g, carry):
        sl = pl.ds(g * L, L)
        acc = cbuf[0, sl] * obuf[0, sl]
        for c in range(1, EMBED):
            acc = acc + cbuf[c, sl] * obuf[c, sl]
        dv[sl] = acc
        return carry

    lax.fori_loop(0, BPW // L, body, 0)
    pltpu.sync_copy(dv, dots_hbm.at[pl.ds(base, BPW)])


def _tc_loss_body(d_ref, o_ref):
    d = d_ref[...]
    neg_abs = -jnp.abs(d)
    ls = jnp.minimum(d, 0.0) - jnp.log(1.0 + jnp.exp(neg_abs))
    o_ref[0, 0] = -jnp.sum(ls) / BATCH


_tc_loss = pl.pallas_call(
    _tc_loss_body,
    out_shape=jax.ShapeDtypeStruct((1, 1), jnp.float32),
    out_specs=pl.BlockSpec(memory_space=pltpu.SMEM),
)


def kernel(x, y, center_weight, out_weight):
    ct = center_weight.T
    ot = out_weight.T
    tpad = ((0, 0), (0, 128 - (VOCAB - (NT - 1) * 128)))
    tcp = jnp.pad(center_weight[(NT - 1) * 128:].T, tpad)
    cf4 = _sc_retile(ct, tcp)
    of4 = _tc_retile(ot)
    cf = cf4.reshape(NGRP * NT * 8 * 128)
    of = of4.reshape(NGRP * NT * 8 * 128)
    x2 = x.reshape(NW * NCH, CH)
    y2 = y.reshape(NW * NCH, CH)
    dots = _sc_dots(x2, y2, cf, of)
    loss = _tc_loss(dots.reshape(BATCH // 128, 128))
    return loss[0, 0]


# speedup vs baseline: 1.2191x; 1.1608x over previous
import functools

import jax
import jax.numpy as jnp
from jax import lax
from jax.experimental import pallas as pl
from jax.experimental.pallas import tpu as pltpu
from jax.experimental.pallas import tpu_sc as plsc

VOCAB = 1000000
EMBED = 32
BATCH = 16384
NC, NS, L = 2, 16, 16
NW = NC * NS
BPW = BATCH // NW
CH = 128
NCH = BPW // CH

NGRP = 4
NT = 7813
GRP_WORDS = NT * 1024
TOTAL_TILES = NGRP * NT
WIN = 16
NWIN = (NT + WIN - 1) // WIN

_mesh = plsc.VectorSubcoreMesh(core_axis_name="c", subcore_axis_name="s")


RWIN = 128
RNB = (NT + RWIN - 1) // RWIN


def _tc_retile_body(o_in, o_out):
    hi = o_in[0:16, :].astype(jnp.bfloat16)
    lo = o_in[16:32, :].astype(jnp.bfloat16)
    hw = lax.bitcast_convert_type(hi, jnp.uint16).astype(jnp.uint32)
    lw = lax.bitcast_convert_type(lo, jnp.uint16).astype(jnp.uint32)
    pw = jnp.bitwise_or(lax.shift_left(hw, jnp.uint32(16)), lw)
    for g in range(NGRP2):
        for i in range(RWIN):
            o_out[g, i] = pw[g * 8:(g + 1) * 8, i * 128:(i + 1) * 128]


NGRP2 = 2

_tc_retile = pl.pallas_call(
    _tc_retile_body,
    grid=(RNB,),
    in_specs=[pl.BlockSpec((32, RWIN * 128), lambda w: (0, w))],
    out_specs=pl.BlockSpec((NGRP2, RWIN, 8, 128), lambda w: (0, w, 0, 0)),
    out_shape=jax.ShapeDtypeStruct((NGRP2, NT, 8, 128), jnp.uint32),
    compiler_params=pltpu.CompilerParams(
        dimension_semantics=("parallel",)),
)

SWIN = 16
SNW = 61


@functools.partial(
    pl.kernel,
    out_type=jax.ShapeDtypeStruct((NGRP, NT, 8, 128), jnp.float32),
    mesh=_mesh,
    compiler_params=pltpu.CompilerParams(needs_layout_passes=False),
    scratch_types=[
        pltpu.VMEM((SWIN, 8, 128), jnp.float32),
        pltpu.VMEM((SWIN, 8, 128), jnp.float32),
        pltpu.VMEM((8, 128), jnp.float32),
        pltpu.SemaphoreType.DMA((2,)),
    ],
)
def _sc_retile(ct_hbm, tc_hbm, cf_hbm, vb0, vb1, tbuf, in_sems):
    wid = lax.axis_index("s") * NC + lax.axis_index("c")
    g = wid % NGRP
    s = wid // NGRP
    t_base = s * (SNW * SWIN)
    vbufs = (vb0, vb1)

    def src_tile(src, t):
        return src.at[pl.ds(pl.multiple_of(g * 8, 8), 8),
                      pl.ds(pl.multiple_of(t * 128, 128), 128)]

    def fire_in(w, slot):
        t0 = t_base + w * SWIN
        for i in range(SWIN):
            pltpu.async_copy(src_tile(ct_hbm, t0 + i),
                             vbufs[slot].at[i], in_sems.at[slot])

    def wait_in(w, slot):
        t0 = t_base + w * SWIN
        for i in range(SWIN):
            pltpu.make_async_copy(src_tile(ct_hbm, t0 + i),
                                  vbufs[slot].at[i],
                                  in_sems.at[slot]).wait()

    def flush_out(w, slot):
        t0 = t_base + w * SWIN
        pltpu.sync_copy(
            vbufs[slot],
            cf_hbm.at[g, pl.ds(pl.multiple_of(t0, SWIN), SWIN)])

    fire_in(0, 0)
    fire_in(1, 1)

    def pair(b, carry):
        for i in range(2):
            w = b * 2 + i
            wait_in(w, i)
            flush_out(w, i)

            @pl.when(w + 2 < SNW)
            def _next():
                fire_in(w + 2, i)
        return carry

    lax.fori_loop(0, SNW // 2, pair, 0)
    wait_in(SNW - 1, (SNW - 1) % 2)
    flush_out(SNW - 1, (SNW - 1) % 2)

    @pl.when(wid < NGRP)
    def _tails():
        for t in range(8 * SNW * SWIN, NT - 1):
            pltpu.sync_copy(src_tile(ct_hbm, t), tbuf)
            pltpu.sync_copy(tbuf, cf_hbm.at[g, t])
        pltpu.sync_copy(
            tc_hbm.at[pl.ds(pl.multiple_of(g * 8, 8), 8), :], tbuf)
        pltpu.sync_copy(tbuf, cf_hbm.at[g, NT - 1])


@functools.partial(
    pl.kernel,
    out_type=jax.ShapeDtypeStruct((BATCH,), jnp.float32),
    mesh=_mesh,
    compiler_params=pltpu.CompilerParams(
        use_tc_tiling_on_sc=False, needs_layout_passes=False),
    scratch_types=[
        pltpu.VMEM((NCH, CH), jnp.int32),
        pltpu.VMEM((NCH, CH), jnp.int32),
        pltpu.VMEM((EMBED, BPW), jnp.float32),
        pltpu.VMEM((EMBED // 2, BPW), jnp.uint32),
        pltpu.VMEM((BPW,), jnp.float32),
        pltpu.SemaphoreType.DMA,
    ],
)
def _sc_dots(x_hbm, y_hbm, cf_hbm, of_hbm, dots_hbm, xp, yp, cbuf, obuf, dv,
             sem):
    wid = lax.axis_index("s") * NC + lax.axis_index("c")
    base = wid * BPW
    pltpu.sync_copy(x_hbm.at[pl.ds(wid * NCH, NCH)], xp)
    pltpu.sync_copy(y_hbm.at[pl.ds(wid * NCH, NCH)], yp)
    for j in range(NCH):
        for k in range(CH // L):
            sl = pl.ds(k * L, L)
            vx = xp[j, sl]
            vy = yp[j, sl]
            xp[j, sl] = lax.shift_left(lax.shift_right_logical(vx, 7), 10) \
                + jnp.bitwise_and(vx, 127)
            yp[j, sl] = lax.shift_left(lax.shift_right_logical(vy, 7), 10) \
                + jnp.bitwise_and(vy, 127)

    def fire(j):
        copies = []
        clen = (NT - 1) * 1024 + 128
        for c in range(EMBED):
            cbase = (c // 8) * GRP_WORDS + (c % 8) * 128
            copies.append(
                pltpu.async_copy(
                    cf_hbm.at[pl.ds(cbase, clen)].at[xp.at[j]],
                    cbuf.at[c, pl.ds(j * CH, CH)], sem))
        for c2 in range(EMBED // 2):
            cbase = (c2 // 8) * GRP_WORDS + (c2 % 8) * 128
            copies.append(
                pltpu.async_copy(
                    of_hbm.at[pl.ds(cbase, clen)].at[yp.at[j]],
                    obuf.at[c2, pl.ds(j * CH, CH)], sem))
        return copies

    pending = fire(0)
    for j in range(NCH):
        nxt = fire(j + 1) if j + 1 < NCH else []
        for cp in pending:
            cp.wait()
        pending = nxt

    himask = jnp.uint32(0xFFFF0000)

    def body(g, carry):
        sl = pl.ds(g * L, L)
        acc = jnp.zeros((L,), jnp.float32)
        for c2 in range(EMBED // 2):
            wy = obuf[c2, sl]
            ayh = plsc.bitcast(jnp.bitwise_and(wy, himask), jnp.float32)
            ayl = plsc.bitcast(lax.shift_left(wy, jnp.uint32(16)), jnp.float32)
            acc = acc + cbuf[c2, sl] * ayh
            acc = acc + cbuf[c2 + EMBED // 2, sl] * ayl
        dv[sl] = acc
        return carry

    lax.fori_loop(0, BPW // L, body, 0)
    pltpu.sync_copy(dv, dots_hbm.at[pl.ds(base, BPW)])


def _tc_loss_body(d_ref, o_ref):
    d = d_ref[...]
    neg_abs = -jnp.abs(d)
    ls = jnp.minimum(d, 0.0) - jnp.log(1.0 + jnp.exp(neg_abs))
    o_ref[0, 0] = -jnp.sum(ls) / BATCH


_tc_loss = pl.pallas_call(
    _tc_loss_body,
    out_shape=jax.ShapeDtypeStruct((1, 1), jnp.float32),
    out_specs=pl.BlockSpec(memory_space=pltpu.SMEM),
)


def kernel(x, y, center_weight, out_weight):
    ct = center_weight.T
    ot = out_weight.T
    tpad = ((0, 0), (0, 128 - (VOCAB - (NT - 1) * 128)))
    tcp = jnp.pad(center_weight[(NT - 1) * 128:].T, tpad)
    cf4 = _sc_retile(ct, tcp)
    of4 = _tc_retile(ot)
    cf = cf4.reshape(NGRP * NT * 8 * 128)
    of = of4.reshape(NGRP2 * NT * 8 * 128)
    x2 = x.reshape(NW * NCH, CH)
    y2 = y.reshape(NW * NCH, CH)
    dots = _sc_dots(x2, y2, cf, of)
    loss = _tc_loss(dots.reshape(BATCH // 128, 128))
    return loss[0, 0]


# speedup vs baseline: 1.4818x; 1.2155x over previous
import functools

import jax
import jax.numpy as jnp
from jax import lax
from jax.experimental import pallas as pl
from jax.experimental.pallas import tpu as pltpu
from jax.experimental.pallas import tpu_sc as plsc

VOCAB = 1000000
EMBED = 32
BATCH = 16384
NC, NS, L = 2, 16, 16
NW = NC * NS
BPW = BATCH // NW
CH = 128
NCH = BPW // CH

NGRP = 4
NT = 7813
GRP_WORDS = NT * 1024
TOTAL_TILES = NGRP * NT
WIN = 16
NWIN = (NT + WIN - 1) // WIN

_mesh = plsc.VectorSubcoreMesh(core_axis_name="c", subcore_axis_name="s")


RWIN = 128
RNB = (NT + RWIN - 1) // RWIN


def _tc_retile_body(c_in, o_in, c_out, o_out):
    for src_ref, dst_ref in ((c_in, c_out), (o_in, o_out)):
        hi = src_ref[0:16, :].astype(jnp.bfloat16)
        lo = src_ref[16:32, :].astype(jnp.bfloat16)
        hw = lax.bitcast_convert_type(hi, jnp.uint16).astype(jnp.uint32)
        lw = lax.bitcast_convert_type(lo, jnp.uint16).astype(jnp.uint32)
        pw = jnp.bitwise_or(lax.shift_left(hw, jnp.uint32(16)), lw)
        for g in range(NGRP2):
            for i in range(RWIN):
                dst_ref[g, i] = pw[g * 8:(g + 1) * 8, i * 128:(i + 1) * 128]


NGRP2 = 2

_tc_retile = pl.pallas_call(
    _tc_retile_body,
    grid=(RNB,),
    in_specs=[
        pl.BlockSpec((32, RWIN * 128), lambda w: (0, w)),
        pl.BlockSpec((32, RWIN * 128), lambda w: (0, w)),
    ],
    out_specs=[
        pl.BlockSpec((NGRP2, RWIN, 8, 128), lambda w: (0, w, 0, 0)),
        pl.BlockSpec((NGRP2, RWIN, 8, 128), lambda w: (0, w, 0, 0)),
    ],
    out_shape=[
        jax.ShapeDtypeStruct((NGRP2, NT, 8, 128), jnp.uint32),
        jax.ShapeDtypeStruct((NGRP2, NT, 8, 128), jnp.uint32),
    ],
    compiler_params=pltpu.CompilerParams(
        dimension_semantics=("parallel",)),
)

SWIN = 16
SNW = 61


@functools.partial(
    pl.kernel,
    out_type=jax.ShapeDtypeStruct((NGRP, NT, 8, 128), jnp.float32),
    mesh=_mesh,
    compiler_params=pltpu.CompilerParams(needs_layout_passes=False),
    scratch_types=[
        pltpu.VMEM((SWIN, 8, 128), jnp.float32),
        pltpu.VMEM((SWIN, 8, 128), jnp.float32),
        pltpu.VMEM((8, 128), jnp.float32),
        pltpu.SemaphoreType.DMA((2,)),
    ],
)
def _sc_retile(ct_hbm, tc_hbm, cf_hbm, vb0, vb1, tbuf, in_sems):
    wid = lax.axis_index("s") * NC + lax.axis_index("c")
    g = wid % NGRP
    s = wid // NGRP
    t_base = s * (SNW * SWIN)
    vbufs = (vb0, vb1)

    def src_tile(src, t):
        return src.at[pl.ds(pl.multiple_of(g * 8, 8), 8),
                      pl.ds(pl.multiple_of(t * 128, 128), 128)]

    def fire_in(w, slot):
        t0 = t_base + w * SWIN
        for i in range(SWIN):
            pltpu.async_copy(src_tile(ct_hbm, t0 + i),
                             vbufs[slot].at[i], in_sems.at[slot])

    def wait_in(w, slot):
        t0 = t_base + w * SWIN
        for i in range(SWIN):
            pltpu.make_async_copy(src_tile(ct_hbm, t0 + i),
                                  vbufs[slot].at[i],
                                  in_sems.at[slot]).wait()

    def flush_out(w, slot):
        t0 = t_base + w * SWIN
        pltpu.sync_copy(
            vbufs[slot],
            cf_hbm.at[g, pl.ds(pl.multiple_of(t0, SWIN), SWIN)])

    fire_in(0, 0)
    fire_in(1, 1)

    def pair(b, carry):
        for i in range(2):
            w = b * 2 + i
            wait_in(w, i)
            flush_out(w, i)

            @pl.when(w + 2 < SNW)
            def _next():
                fire_in(w + 2, i)
        return carry

    lax.fori_loop(0, SNW // 2, pair, 0)
    wait_in(SNW - 1, (SNW - 1) % 2)
    flush_out(SNW - 1, (SNW - 1) % 2)

    @pl.when(wid < NGRP)
    def _tails():
        for t in range(8 * SNW * SWIN, NT - 1):
            pltpu.sync_copy(src_tile(ct_hbm, t), tbuf)
            pltpu.sync_copy(tbuf, cf_hbm.at[g, t])
        pltpu.sync_copy(
            tc_hbm.at[pl.ds(pl.multiple_of(g * 8, 8), 8), :], tbuf)
        pltpu.sync_copy(tbuf, cf_hbm.at[g, NT - 1])


@functools.partial(
    pl.kernel,
    out_type=jax.ShapeDtypeStruct((BATCH,), jnp.float32),
    mesh=_mesh,
    compiler_params=pltpu.CompilerParams(
        use_tc_tiling_on_sc=False, needs_layout_passes=False),
    scratch_types=[
        pltpu.VMEM((NCH, CH), jnp.int32),
        pltpu.VMEM((NCH, CH), jnp.int32),
        pltpu.VMEM((EMBED // 2, BPW), jnp.uint32),
        pltpu.VMEM((EMBED // 2, BPW), jnp.uint32),
        pltpu.VMEM((BPW,), jnp.float32),
        pltpu.SemaphoreType.DMA,
    ],
)
def _sc_dots(x_hbm, y_hbm, cf_hbm, of_hbm, dots_hbm, xp, yp, cbuf, obuf, dv,
             sem):
    wid = lax.axis_index("s") * NC + lax.axis_index("c")
    base = wid * BPW
    pltpu.sync_copy(x_hbm.at[pl.ds(wid * NCH, NCH)], xp)
    pltpu.sync_copy(y_hbm.at[pl.ds(wid * NCH, NCH)], yp)
    for j in range(NCH):
        for k in range(CH // L):
            sl = pl.ds(k * L, L)
            vx = xp[j, sl]
            vy = yp[j, sl]
            xp[j, sl] = lax.shift_left(lax.shift_right_logical(vx, 7), 10) \
                + jnp.bitwise_and(vx, 127)
            yp[j, sl] = lax.shift_left(lax.shift_right_logical(vy, 7), 10) \
                + jnp.bitwise_and(vy, 127)

    def fire(j):
        copies = []
        clen = (NT - 1) * 1024 + 128
        for c2 in range(EMBED // 2):
            cbase = (c2 // 8) * GRP_WORDS + (c2 % 8) * 128
            copies.append(
                pltpu.async_copy(
                    cf_hbm.at[pl.ds(cbase, clen)].at[xp.at[j]],
                    cbuf.at[c2, pl.ds(j * CH, CH)], sem))
            copies.append(
                pltpu.async_copy(
                    of_hbm.at[pl.ds(cbase, clen)].at[yp.at[j]],
                    obuf.at[c2, pl.ds(j * CH, CH)], sem))
        return copies

    pending = fire(0)
    for j in range(NCH):
        nxt = fire(j + 1) if j + 1 < NCH else []
        for cp in pending:
            cp.wait()
        pending = nxt

    himask = jnp.uint32(0xFFFF0000)

    def body(g, carry):
        sl = pl.ds(g * L, L)
        acc = jnp.zeros((L,), jnp.float32)
        for c2 in range(EMBED // 2):
            wx = cbuf[c2, sl]
            wy = obuf[c2, sl]
            axh = plsc.bitcast(jnp.bitwise_and(wx, himask), jnp.float32)
            ayh = plsc.bitcast(jnp.bitwise_and(wy, himask), jnp.float32)
            axl = plsc.bitcast(lax.shift_left(wx, jnp.uint32(16)), jnp.float32)
            ayl = plsc.bitcast(lax.shift_left(wy, jnp.uint32(16)), jnp.float32)
            acc = acc + axh * ayh
            acc = acc + axl * ayl
        dv[sl] = acc
        return carry

    lax.fori_loop(0, BPW // L, body, 0)
    pltpu.sync_copy(dv, dots_hbm.at[pl.ds(base, BPW)])


def _tc_loss_body(d_ref, o_ref):
    d = d_ref[...]
    neg_abs = -jnp.abs(d)
    ls = jnp.minimum(d, 0.0) - jnp.log(1.0 + jnp.exp(neg_abs))
    o_ref[0, 0] = -jnp.sum(ls) / BATCH


_tc_loss = pl.pallas_call(
    _tc_loss_body,
    out_shape=jax.ShapeDtypeStruct((1, 1), jnp.float32),
    out_specs=pl.BlockSpec(memory_space=pltpu.SMEM),
)


def kernel(x, y, center_weight, out_weight):
    ct = center_weight.T
    ot = out_weight.T
    cf4, of4 = _tc_retile(ct, ot)
    cf = cf4.reshape(NGRP2 * NT * 8 * 128)
    of = of4.reshape(NGRP2 * NT * 8 * 128)
    x2 = x.reshape(NW * NCH, CH)
    y2 = y.reshape(NW * NCH, CH)
    dots = _sc_dots(x2, y2, cf, of)
    loss = _tc_loss(dots.reshape(BATCH // 128, 128))
    return loss[0, 0]


# speedup vs baseline: 1.4829x; 1.0007x over previous
import functools

import jax
import jax.numpy as jnp
from jax import lax
from jax.experimental import pallas as pl
from jax.experimental.pallas import tpu as pltpu
from jax.experimental.pallas import tpu_sc as plsc

VOCAB = 1000000
EMBED = 32
BATCH = 16384
NC, NS, L = 2, 16, 16
NW = NC * NS
BPW = BATCH // NW
CH = 128
NCH = BPW // CH

NGRP = 4
NT = 7813
GRP_WORDS = NT * 1024

_mesh = plsc.VectorSubcoreMesh(core_axis_name="c", subcore_axis_name="s")


RWIN = 128
RNB = (NT + RWIN - 1) // RWIN


def _tc_retile_body(c_in, o_in, c_out, o_out):
    for src_ref, dst_ref in ((c_in, c_out), (o_in, o_out)):
        hi = src_ref[0:16, :].astype(jnp.bfloat16)
        lo = src_ref[16:32, :].astype(jnp.bfloat16)
        hw = lax.bitcast_convert_type(hi, jnp.uint16).astype(jnp.uint32)
        lw = lax.bitcast_convert_type(lo, jnp.uint16).astype(jnp.uint32)
        pw = jnp.bitwise_or(lax.shift_left(hw, jnp.uint32(16)), lw)
        for g in range(NGRP2):
            for i in range(RWIN):
                dst_ref[g, i] = pw[g * 8:(g + 1) * 8, i * 128:(i + 1) * 128]


NGRP2 = 2

_tc_retile = pl.pallas_call(
    _tc_retile_body,
    grid=(RNB,),
    in_specs=[
        pl.BlockSpec((32, RWIN * 128), lambda w: (0, w)),
        pl.BlockSpec((32, RWIN * 128), lambda w: (0, w)),
    ],
    out_specs=[
        pl.BlockSpec((NGRP2, RWIN, 8, 128), lambda w: (0, w, 0, 0)),
        pl.BlockSpec((NGRP2, RWIN, 8, 128), lambda w: (0, w, 0, 0)),
    ],
    out_shape=[
        jax.ShapeDtypeStruct((NGRP2, NT, 8, 128), jnp.uint32),
        jax.ShapeDtypeStruct((NGRP2, NT, 8, 128), jnp.uint32),
    ],
    compiler_params=pltpu.CompilerParams(
        dimension_semantics=("parallel",)),
)

@functools.partial(
    pl.kernel,
    out_type=jax.ShapeDtypeStruct((BATCH,), jnp.float32),
    mesh=_mesh,
    compiler_params=pltpu.CompilerParams(
        use_tc_tiling_on_sc=False, needs_layout_passes=False),
    scratch_types=[
        pltpu.VMEM((NCH, CH), jnp.int32),
        pltpu.VMEM((NCH, CH), jnp.int32),
        pltpu.VMEM((EMBED // 2, BPW), jnp.uint32),
        pltpu.VMEM((EMBED // 2, BPW), jnp.uint32),
        pltpu.VMEM((BPW,), jnp.float32),
        pltpu.SemaphoreType.DMA,
    ],
)
def _sc_dots(x_hbm, y_hbm, cf_hbm, of_hbm, dots_hbm, xp, yp, cbuf, obuf, dv,
             sem):
    wid = lax.axis_index("s") * NC + lax.axis_index("c")
    base = wid * BPW
    pltpu.sync_copy(x_hbm.at[pl.ds(wid * NCH, NCH)], xp)
    pltpu.sync_copy(y_hbm.at[pl.ds(wid * NCH, NCH)], yp)
    for j in range(NCH):
        for k in range(CH // L):
            sl = pl.ds(k * L, L)
            vx = xp[j, sl]
            vy = yp[j, sl]
            xp[j, sl] = lax.shift_left(lax.shift_right_logical(vx, 7), 10) \
                + jnp.bitwise_and(vx, 127)
            yp[j, sl] = lax.shift_left(lax.shift_right_logical(vy, 7), 10) \
                + jnp.bitwise_and(vy, 127)

    def fire(j):
        copies = []
        clen = (NT - 1) * 1024 + 128
        for c2 in range(EMBED // 2):
            cbase = (c2 // 8) * GRP_WORDS + (c2 % 8) * 128
            copies.append(
                pltpu.async_copy(
                    cf_hbm.at[pl.ds(cbase, clen)].at[xp.at[j]],
                    cbuf.at[c2, pl.ds(j * CH, CH)], sem))
            copies.append(
                pltpu.async_copy(
                    of_hbm.at[pl.ds(cbase, clen)].at[yp.at[j]],
                    obuf.at[c2, pl.ds(j * CH, CH)], sem))
        return copies

    pending = fire(0)
    for j in range(NCH):
        nxt = fire(j + 1) if j + 1 < NCH else []
        for cp in pending:
            cp.wait()
        pending = nxt

    himask = jnp.uint32(0xFFFF0000)

    def body(g, carry):
        sl = pl.ds(g * L, L)
        acc = jnp.zeros((L,), jnp.float32)
        for c2 in range(EMBED // 2):
            wx = cbuf[c2, sl]
            wy = obuf[c2, sl]
            axh = plsc.bitcast(jnp.bitwise_and(wx, himask), jnp.float32)
            ayh = plsc.bitcast(jnp.bitwise_and(wy, himask), jnp.float32)
            axl = plsc.bitcast(lax.shift_left(wx, jnp.uint32(16)), jnp.float32)
            ayl = plsc.bitcast(lax.shift_left(wy, jnp.uint32(16)), jnp.float32)
            acc = acc + axh * ayh
            acc = acc + axl * ayl
        dv[sl] = acc
        return carry

    lax.fori_loop(0, BPW // L, body, 0)
    pltpu.sync_copy(dv, dots_hbm.at[pl.ds(base, BPW)])


def _tc_loss_body(d_ref, o_ref):
    d = d_ref[...]
    neg_abs = -jnp.abs(d)
    ls = jnp.minimum(d, 0.0) - jnp.log(1.0 + jnp.exp(neg_abs))
    o_ref[0, 0] = -jnp.sum(ls) / BATCH


_tc_loss = pl.pallas_call(
    _tc_loss_body,
    out_shape=jax.ShapeDtypeStruct((1, 1), jnp.float32),
    out_specs=pl.BlockSpec(memory_space=pltpu.SMEM),
)


def kernel(x, y, center_weight, out_weight):
    ct = center_weight.T
    ot = out_weight.T
    cf4, of4 = _tc_retile(ct, ot)
    cf = cf4.reshape(NGRP2 * NT * 8 * 128)
    of = of4.reshape(NGRP2 * NT * 8 * 128)
    x2 = x.reshape(NW * NCH, CH)
    y2 = y.reshape(NW * NCH, CH)
    dots = _sc_dots(x2, y2, cf, of)
    loss = _tc_loss(dots.reshape(BATCH // 128, 128))
    return loss[0, 0]
